# baseline (speedup 1.0000x reference)
import functools

import numpy as np
import jax
import jax.numpy as jnp
from jax import lax
from jax.experimental import pallas as pl
from jax.experimental.pallas import tpu as pltpu
from jax.experimental.pallas import tpu_sc as plsc

_N = 50000
_NEIGH = 32
_F = 16
_NW = 32
_NC = 2
_CHUNK = 64
_CPW = 25
_PERW = _CHUNK * _CPW
_NPAD = _NW * _PERW
_EDGES_PAD = _NPAD * _NEIGH
_CE = _CHUNK * _NEIGH
_EB = _CHUNK * 4 * _NEIGH
_RB = 512
_RU = 512

_peak_std = np.ones(16, dtype=np.float32)
_peak_avg = np.zeros(16, dtype=np.float32)
for _k, _v in {1: (0, 4.8, 2.8), 6: (0, 100.0, 50.0), 7: (0, 120.0, 80.0)}.items():
    _peak_std[_k] = _v[2]
    _peak_avg[_k] = _v[1]


def _edge_body(w0_ref, b0_ref, w1_ref, b1_ref, x_ref, o_ref):
    x = x_ref[...]
    mask = (x > 1e-5).astype(jnp.float32)
    cs = np.linspace(0.0, 1.0, 4).astype(np.float32)
    gap2 = np.float32(cs[1] - cs[0]) ** 2
    rbf = [jnp.exp(-((x - cs[c]) ** 2) / gap2) * mask for c in range(4)]
    h = [
        jnp.maximum(
            rbf[0] * w0_ref[0, m] + rbf[1] * w0_ref[1, m]
            + rbf[2] * w0_ref[2, m] + rbf[3] * w0_ref[3, m] + b0_ref[m],
            0.0,
        )
        for m in range(4)
    ]
    for k in range(4):
        g = jnp.tanh(
            h[0] * w1_ref[0, k] + h[1] * w1_ref[1, k]
            + h[2] * w1_ref[2, k] + h[3] * w1_ref[3, k] + b1_ref[k]
        )
        o_ref[:, k * _NEIGH:(k + 1) * _NEIGH] = g


def _edge_call(edge_p, w0, b0, w1, b1):
    return pl.pallas_call(
        _edge_body,
        grid=(_NPAD // _RB,),
        in_specs=[
            pl.BlockSpec(memory_space=pltpu.SMEM),
            pl.BlockSpec(memory_space=pltpu.SMEM),
            pl.BlockSpec(memory_space=pltpu.SMEM),
            pl.BlockSpec(memory_space=pltpu.SMEM),
            pl.BlockSpec((_RB, _NEIGH), lambda i: (i, 0)),
        ],
        out_specs=pl.BlockSpec((_RB, 128), lambda i: (i, 0)),
        out_shape=jax.ShapeDtypeStruct((_NPAD, 128), jnp.float32),
    )(w0, b0, w1, b1, edge_p)


def _mp_body(nodes_hbm, nlist_hbm, e_hbm, w_hbm, inv_hbm, out_hbm,
             idx_v, rows_v, e_v, own_v, inv_v, out_v, w_v, sem_a, sem_b, sem_g):
    wid = lax.axis_index("s") * _NC + lax.axis_index("c")

    pltpu.sync_copy(w_hbm, w_v)

    def issue_ie(ci, sem):
        cid = wid * _CPW + ci
        slot = jnp.bitwise_and(ci, 3)
        slot2 = slot
        base = cid * _CHUNK
        pltpu.async_copy(
            nlist_hbm.at[pl.ds(cid * 16, 16)], idx_v.at[pl.ds(slot * 16, 16)], sem
        )
        pltpu.async_copy(
            e_hbm.at[pl.ds(cid * _EB, _EB)], e_v.at[pl.ds(slot * _EB, _EB)], sem
        )
        pltpu.async_copy(
            nodes_hbm.at[pl.ds(base, _CHUNK)],
            own_v.at[pl.ds(slot2 * _CHUNK, _CHUNK)], sem
        )
        pltpu.async_copy(
            inv_hbm.at[pl.ds(base, _CHUNK)],
            inv_v.at[pl.ds(slot2 * _CHUNK, _CHUNK)], sem
        )

    def wait_ie(sem):
        pltpu.make_async_copy(
            nlist_hbm.at[pl.ds(0, 16)], idx_v.at[pl.ds(0, 16)], sem
        ).wait()
        pltpu.make_async_copy(
            e_hbm.at[pl.ds(0, _EB)], e_v.at[pl.ds(0, _EB)], sem
        ).wait()
        pltpu.make_async_copy(
            nodes_hbm.at[pl.ds(0, _CHUNK)], own_v.at[pl.ds(0, _CHUNK)], sem
        ).wait()
        pltpu.make_async_copy(
            inv_hbm.at[pl.ds(0, _CHUNK)], inv_v.at[pl.ds(0, _CHUNK)], sem
        ).wait()

    def start_gather(ci):
        islot = jnp.bitwise_and(ci, 3)
        rslot = jnp.bitwise_and(ci, 1)
        for r in range(16):
            pltpu.async_copy(
                nodes_hbm.at[idx_v.at[islot * 16 + r]],
                rows_v.at[pl.ds(rslot * _CE + r * 128, 128)],
                sem_g,
            )

    issue_ie(0, sem_a)
    issue_ie(1, sem_b)
    wait_ie(sem_a)
    start_gather(0)

    @pl.loop(0, _CPW)
    def _chunk(c):
        par = jnp.bitwise_and(c, 1)
        base = (wid * _CPW + c) * _CHUNK
        slot = jnp.bitwise_and(c, 3)
        rslot = par

        @pl.when(jnp.logical_and(c + 2 < _CPW, par == 0))
        def _():
            issue_ie(c + 2, sem_a)

        @pl.when(jnp.logical_and(c + 2 < _CPW, par == 1))
        def _():
            issue_ie(c + 2, sem_b)

        @pl.when(jnp.logical_and(c + 1 < _CPW, par == 0))
        def _():
            wait_ie(sem_b)

        @pl.when(jnp.logical_and(c + 1 < _CPW, par == 1))
        def _():
            wait_ie(sem_a)

        pltpu.make_async_copy(
            nodes_hbm.at[pl.ds(0, _CE)], rows_v.at[pl.ds(rslot * _CE, _CE)], sem_g
        ).wait()

        @pl.when(c + 1 < _CPW)
        def _():
            start_gather(c + 1)

        @plsc.parallel_loop(0, _CHUNK, unroll=4)
        def _node(i):
            e0 = slot * _EB + i * 128
            f0 = rslot * _CE + i * _NEIGH
            evs = [
                (e_v[pl.ds(e0 + k * _NEIGH, 16)], e_v[pl.ds(e0 + k * _NEIGH + 16, 16)])
                for k in range(4)
            ]
            accs = [jnp.zeros((16,), jnp.float32) for _ in range(4)]
            for j in range(_NEIGH):
                row = rows_v[f0 + j]
                lane = jnp.full((16,), j % 16, jnp.int32)
                for k in range(4):
                    src = evs[k][0] if j < 16 else evs[k][1]
                    ev = jnp.take_along_axis(src, lane, axis=0)
                    accs[k] = accs[k] + row * ev
            red = jnp.zeros((16,), jnp.float32)
            for k in range(4):
                for n in range(16):
                    lane_n = jnp.full((16,), n, jnp.int32)
                    mt = jnp.take_along_axis(accs[k], lane_n, axis=0)
                    red = red + mt * w_v[pl.ds((k * 16 + n) * 16, 16)]
            iseg = inv_v[pl.ds(slot * _CHUNK + (i & ~15), 16)]
            binv = jnp.take_along_axis(iseg, jnp.bitwise_and(i, 15) + jnp.zeros((16,), jnp.int32), axis=0)
            out_v[i] = jnp.maximum(red * binv, 0.0) + own_v[slot * _CHUNK + i]

        pltpu.sync_copy(out_v, out_hbm.at[pl.ds(base, _CHUNK)])


@functools.lru_cache(maxsize=None)
def _get_mp_kernel():
    mesh = plsc.VectorSubcoreMesh(core_axis_name="c", subcore_axis_name="s")
    return pl.kernel(
        _mp_body,
        out_type=jax.ShapeDtypeStruct((_NPAD, _F), jnp.float32),
        mesh=mesh,
        compiler_params=pltpu.CompilerParams(use_tc_tiling_on_sc=False),
        scratch_types=[
            pltpu.VMEM((4 * 16, 128), jnp.int32),
            pltpu.VMEM((2 * _CE, _F), jnp.float32),
            pltpu.VMEM((4 * _EB,), jnp.float32),
            pltpu.VMEM((4 * _CHUNK, _F), jnp.float32),
            pltpu.VMEM((4 * _CHUNK,), jnp.float32),
            pltpu.VMEM((_CHUNK, _F), jnp.float32),
            pltpu.VMEM((64 * 16,), jnp.float32),
            pltpu.SemaphoreType.DMA,
            pltpu.SemaphoreType.DMA,
            pltpu.SemaphoreType.DMA,
        ],
    )


def _fc_body(w0, b0, w1, b1, w2, b2, ow, ob, pstd, pavg, nodes_ref, ninp_ref, o_ref):
    h = jnp.maximum(jnp.dot(nodes_ref[...], w0[...], preferred_element_type=jnp.float32) + b0[...], 0.0)
    h = jnp.maximum(jnp.dot(h, w1[...], preferred_element_type=jnp.float32) + b1[...], 0.0)
    h = jnp.tanh(jnp.dot(h, w2[...], preferred_element_type=jnp.float32) + b2[...])
    fp = jnp.dot(h, ow[...], preferred_element_type=jnp.float32) + ob[...]
    ni = ninp_ref[...]
    o_ref[...] = jnp.sum(fp * ni * pstd[...] + ni * pavg[...], axis=1, keepdims=True)


def _fc_call(nodes, ninp, w0, b0, w1, b1, w2, b2, ow, ob, pstd, pavg):
    grid = (_N + _RU - 1) // _RU
    full = lambda a, b: pl.BlockSpec((a, b), lambda i: (0, 0))
    return pl.pallas_call(
        _fc_body,
        grid=(grid,),
        in_specs=[
            full(16, 128), full(1, 128),
            full(128, 128), full(1, 128),
            full(128, 128), full(1, 128),
            full(128, 16), full(1, 16),
            full(1, 16), full(1, 16),
            pl.BlockSpec((_RU, _F), lambda i: (i, 0)),
            pl.BlockSpec((_RU, _F), lambda i: (i, 0)),
        ],
        out_specs=pl.BlockSpec((_RU, 1), lambda i: (i, 0)),
        out_shape=jax.ShapeDtypeStruct((_N, 1), jnp.float32),
    )(w0, b0, w1, b1, w2, b2, ow, ob, pstd, pavg, nodes, ninp)


def kernel(node_input, nlist_input, edge_input, inv_degree, edge_W0, edge_b0,
           edge_W1, edge_b1, mp_w0, mp_w1, mp_w2, mp_w3, fc_W0, fc_b0, fc_W1,
           fc_b1, fc_W2, fc_b2, out_W, out_b):
    pad = _NPAD - _N
    edge_p = jnp.pad(edge_input, ((0, pad), (0, 0)))
    nlist2d = (
        jnp.pad(nlist_input.astype(jnp.int32), ((0, pad), (0, 0)))
        .reshape(_EDGES_PAD // 128, 128)
    )
    inv_p = jnp.pad(inv_degree, (0, pad))
    pstd = jnp.asarray(_peak_std).reshape(1, 16)
    pavg = jnp.asarray(_peak_avg).reshape(1, 16)

    e_flat = _edge_call(edge_p, edge_W0, edge_b0, edge_W1, edge_b1).reshape(-1)

    nodes = jnp.pad(node_input, ((0, pad), (0, 0)))
    for w in (mp_w0, mp_w1, mp_w2, mp_w3):
        w_flat = w.transpose(1, 0, 2).reshape(-1)
        nodes = _get_mp_kernel()(nodes, nlist2d, e_flat, w_flat, inv_p)

    peaks = _fc_call(
        nodes, node_input, fc_W0, fc_b0.reshape(1, 128), fc_W1,
        fc_b1.reshape(1, 128), fc_W2, fc_b2.reshape(1, 128), out_W,
        out_b.reshape(1, 16), pstd, pavg,
    )
    return peaks.reshape(_N)


# speedup vs baseline: 1.0103x; 1.0103x over previous
import functools

import numpy as np
import jax
import jax.numpy as jnp
from jax import lax
from jax.experimental import pallas as pl
from jax.experimental.pallas import tpu as pltpu
from jax.experimental.pallas import tpu_sc as plsc

_N = 50000
_NEIGH = 32
_F = 16
_NW = 32
_NC = 2
_CHUNK = 64
_CPW = 25
_PERW = _CHUNK * _CPW
_NPAD = _NW * _PERW
_EDGES_PAD = _NPAD * _NEIGH
_CE = _CHUNK * _NEIGH
_EB = _CHUNK * 4 * _NEIGH
_RB = 512
_RU = 512

_peak_std = np.ones(16, dtype=np.float32)
_peak_avg = np.zeros(16, dtype=np.float32)
for _k, _v in {1: (0, 4.8, 2.8), 6: (0, 100.0, 50.0), 7: (0, 120.0, 80.0)}.items():
    _peak_std[_k] = _v[2]
    _peak_avg[_k] = _v[1]


def _edge_body(w0_ref, b0_ref, w1_ref, b1_ref, x_ref, o_ref):
    x = x_ref[...]
    mask = (x > 1e-5).astype(jnp.float32)
    cs = np.linspace(0.0, 1.0, 4).astype(np.float32)
    gap2 = np.float32(cs[1] - cs[0]) ** 2
    rbf = [jnp.exp(-((x - cs[c]) ** 2) / gap2) * mask for c in range(4)]
    h = [
        jnp.maximum(
            rbf[0] * w0_ref[0, m] + rbf[1] * w0_ref[1, m]
            + rbf[2] * w0_ref[2, m] + rbf[3] * w0_ref[3, m] + b0_ref[m],
            0.0,
        )
        for m in range(4)
    ]
    for k in range(4):
        g = jnp.tanh(
            h[0] * w1_ref[0, k] + h[1] * w1_ref[1, k]
            + h[2] * w1_ref[2, k] + h[3] * w1_ref[3, k] + b1_ref[k]
        )
        o_ref[:, k * _NEIGH:(k + 1) * _NEIGH] = g


def _edge_call(edge_p, w0, b0, w1, b1):
    return pl.pallas_call(
        _edge_body,
        grid=(_NPAD // _RB,),
        in_specs=[
            pl.BlockSpec(memory_space=pltpu.SMEM),
            pl.BlockSpec(memory_space=pltpu.SMEM),
            pl.BlockSpec(memory_space=pltpu.SMEM),
            pl.BlockSpec(memory_space=pltpu.SMEM),
            pl.BlockSpec((_RB, _NEIGH), lambda i: (i, 0)),
        ],
        out_specs=pl.BlockSpec((_RB, 128), lambda i: (i, 0)),
        out_shape=jax.ShapeDtypeStruct((_NPAD, 128), jnp.float32),
    )(w0, b0, w1, b1, edge_p)


def _mp_body(nodes_hbm, nlist_hbm, e_hbm, w_hbm, inv_hbm, out_hbm,
             idx_v, rows_v, e_v, own_v, inv_v, out_v, w_v, sem_a, sem_b, sem_g):
    wid = lax.axis_index("s") * _NC + lax.axis_index("c")

    pltpu.sync_copy(w_hbm, w_v)

    def issue_ie(ci, sem):
        cid = wid * _CPW + ci
        slot = jnp.bitwise_and(ci, 3)
        slot2 = slot
        base = cid * _CHUNK
        pltpu.async_copy(
            nlist_hbm.at[pl.ds(cid * 16, 16)], idx_v.at[pl.ds(slot * 16, 16)], sem
        )
        pltpu.async_copy(
            e_hbm.at[pl.ds(cid * _EB, _EB)], e_v.at[pl.ds(slot * _EB, _EB)], sem
        )
        pltpu.async_copy(
            nodes_hbm.at[pl.ds(base, _CHUNK)],
            own_v.at[pl.ds(slot2 * _CHUNK, _CHUNK)], sem
        )
        pltpu.async_copy(
            inv_hbm.at[pl.ds(base, _CHUNK)],
            inv_v.at[pl.ds(slot2 * _CHUNK, _CHUNK)], sem
        )

    def wait_ie(sem):
        pltpu.make_async_copy(
            nlist_hbm.at[pl.ds(0, 16)], idx_v.at[pl.ds(0, 16)], sem
        ).wait()
        pltpu.make_async_copy(
            e_hbm.at[pl.ds(0, _EB)], e_v.at[pl.ds(0, _EB)], sem
        ).wait()
        pltpu.make_async_copy(
            nodes_hbm.at[pl.ds(0, _CHUNK)], own_v.at[pl.ds(0, _CHUNK)], sem
        ).wait()
        pltpu.make_async_copy(
            inv_hbm.at[pl.ds(0, _CHUNK)], inv_v.at[pl.ds(0, _CHUNK)], sem
        ).wait()

    def start_gather(ci):
        islot = jnp.bitwise_and(ci, 3)
        rslot = jnp.bitwise_and(ci, 1)
        for r in range(16):
            pltpu.async_copy(
                nodes_hbm.at[idx_v.at[islot * 16 + r]],
                rows_v.at[pl.ds(rslot * _CE + r * 128, 128)],
                sem_g,
            )

    issue_ie(0, sem_a)
    issue_ie(1, sem_b)
    wait_ie(sem_a)
    start_gather(0)

    @pl.loop(0, _CPW)
    def _chunk(c):
        par = jnp.bitwise_and(c, 1)
        base = (wid * _CPW + c) * _CHUNK
        slot = jnp.bitwise_and(c, 3)
        rslot = par

        @pl.when(jnp.logical_and(c + 2 < _CPW, par == 0))
        def _():
            issue_ie(c + 2, sem_a)

        @pl.when(jnp.logical_and(c + 2 < _CPW, par == 1))
        def _():
            issue_ie(c + 2, sem_b)

        @pl.when(jnp.logical_and(c + 1 < _CPW, par == 0))
        def _():
            wait_ie(sem_b)

        @pl.when(jnp.logical_and(c + 1 < _CPW, par == 1))
        def _():
            wait_ie(sem_a)

        pltpu.make_async_copy(
            nodes_hbm.at[pl.ds(0, _CE)], rows_v.at[pl.ds(rslot * _CE, _CE)], sem_g
        ).wait()

        @pl.when(c + 1 < _CPW)
        def _():
            start_gather(c + 1)

        @plsc.parallel_loop(0, _CHUNK, unroll=2)
        def _node(i):
            e0 = slot * _EB + i * 128
            f0 = rslot * _CE + i * _NEIGH
            evs = [
                (e_v[pl.ds(e0 + k * _NEIGH, 16)], e_v[pl.ds(e0 + k * _NEIGH + 16, 16)])
                for k in range(4)
            ]
            accs = [jnp.zeros((16,), jnp.float32) for _ in range(4)]
            for j in range(_NEIGH):
                row = rows_v[f0 + j]
                lane = jnp.full((16,), j % 16, jnp.int32)
                for k in range(4):
                    src = evs[k][0] if j < 16 else evs[k][1]
                    ev = jnp.take_along_axis(src, lane, axis=0)
                    accs[k] = accs[k] + row * ev
            red = jnp.zeros((16,), jnp.float32)
            for k in range(4):
                for n in range(16):
                    lane_n = jnp.full((16,), n, jnp.int32)
                    mt = jnp.take_along_axis(accs[k], lane_n, axis=0)
                    red = red + mt * w_v[pl.ds((k * 16 + n) * 16, 16)]
            iseg = inv_v[pl.ds(slot * _CHUNK + (i & ~15), 16)]
            binv = jnp.take_along_axis(iseg, jnp.bitwise_and(i, 15) + jnp.zeros((16,), jnp.int32), axis=0)
            out_v[i] = jnp.maximum(red * binv, 0.0) + own_v[slot * _CHUNK + i]

        pltpu.sync_copy(out_v, out_hbm.at[pl.ds(base, _CHUNK)])


@functools.lru_cache(maxsize=None)
def _get_mp_kernel():
    mesh = plsc.VectorSubcoreMesh(core_axis_name="c", subcore_axis_name="s")
    return pl.kernel(
        _mp_body,
        out_type=jax.ShapeDtypeStruct((_NPAD, _F), jnp.float32),
        mesh=mesh,
        compiler_params=pltpu.CompilerParams(use_tc_tiling_on_sc=False),
        scratch_types=[
            pltpu.VMEM((4 * 16, 128), jnp.int32),
            pltpu.VMEM((2 * _CE, _F), jnp.float32),
            pltpu.VMEM((4 * _EB,), jnp.float32),
            pltpu.VMEM((4 * _CHUNK, _F), jnp.float32),
            pltpu.VMEM((4 * _CHUNK,), jnp.float32),
            pltpu.VMEM((_CHUNK, _F), jnp.float32),
            pltpu.VMEM((64 * 16,), jnp.float32),
            pltpu.SemaphoreType.DMA,
            pltpu.SemaphoreType.DMA,
            pltpu.SemaphoreType.DMA,
        ],
    )


def _fc_body(w0, b0, w1, b1, w2, b2, ow, ob, pstd, pavg, nodes_ref, ninp_ref, o_ref):
    h = jnp.maximum(jnp.dot(nodes_ref[...], w0[...], preferred_element_type=jnp.float32) + b0[...], 0.0)
    h = jnp.maximum(jnp.dot(h, w1[...], preferred_element_type=jnp.float32) + b1[...], 0.0)
    h = jnp.tanh(jnp.dot(h, w2[...], preferred_element_type=jnp.float32) + b2[...])
    fp = jnp.dot(h, ow[...], preferred_element_type=jnp.float32) + ob[...]
    ni = ninp_ref[...]
    o_ref[...] = jnp.sum(fp * ni * pstd[...] + ni * pavg[...], axis=1, keepdims=True)


def _fc_call(nodes, ninp, w0, b0, w1, b1, w2, b2, ow, ob, pstd, pavg):
    grid = (_N + _RU - 1) // _RU
    full = lambda a, b: pl.BlockSpec((a, b), lambda i: (0, 0))
    return pl.pallas_call(
        _fc_body,
        grid=(grid,),
        in_specs=[
            full(16, 128), full(1, 128),
            full(128, 128), full(1, 128),
            full(128, 128), full(1, 128),
            full(128, 16), full(1, 16),
            full(1, 16), full(1, 16),
            pl.BlockSpec((_RU, _F), lambda i: (i, 0)),
            pl.BlockSpec((_RU, _F), lambda i: (i, 0)),
        ],
        out_specs=pl.BlockSpec((_RU, 1), lambda i: (i, 0)),
        out_shape=jax.ShapeDtypeStruct((_N, 1), jnp.float32),
    )(w0, b0, w1, b1, w2, b2, ow, ob, pstd, pavg, nodes, ninp)


def kernel(node_input, nlist_input, edge_input, inv_degree, edge_W0, edge_b0,
           edge_W1, edge_b1, mp_w0, mp_w1, mp_w2, mp_w3, fc_W0, fc_b0, fc_W1,
           fc_b1, fc_W2, fc_b2, out_W, out_b):
    pad = _NPAD - _N
    edge_p = jnp.pad(edge_input, ((0, pad), (0, 0)))
    nlist2d = (
        jnp.pad(nlist_input.astype(jnp.int32), ((0, pad), (0, 0)))
        .reshape(_EDGES_PAD // 128, 128)
    )
    inv_p = jnp.pad(inv_degree, (0, pad))
    pstd = jnp.asarray(_peak_std).reshape(1, 16)
    pavg = jnp.asarray(_peak_avg).reshape(1, 16)

    e_flat = _edge_call(edge_p, edge_W0, edge_b0, edge_W1, edge_b1).reshape(-1)

    nodes = jnp.pad(node_input, ((0, pad), (0, 0)))
    for w in (mp_w0, mp_w1, mp_w2, mp_w3):
        w_flat = w.transpose(1, 0, 2).reshape(-1)
        nodes = _get_mp_kernel()(nodes, nlist2d, e_flat, w_flat, inv_p)

    peaks = _fc_call(
        nodes, node_input, fc_W0, fc_b0.reshape(1, 128), fc_W1,
        fc_b1.reshape(1, 128), fc_W2, fc_b2.reshape(1, 128), out_W,
        out_b.reshape(1, 16), pstd, pavg,
    )
    return peaks.reshape(_N)
